# baseline (speedup 1.0000x reference)
import functools

import jax
import jax.numpy as jnp
from jax.experimental import pallas as pl
from jax.experimental.pallas import tpu as pltpu

O, P, G, N, H, D = 3, 128, 64, 20000, 512, 128
TEMP_INV = 10.0
EPS = 1e-12
SLAB = H // 128
GROUPS = P // 8


def _proj_kernel(idx_ref, emb_ref, W1_ref, b1_ref, W2_ref, b2_ref,
                 z_ref, emb_vmem, protos, sem):
    o = pl.program_id(0)
    cp = pltpu.make_async_copy(emb_ref.at[o], emb_vmem, sem)
    cp.start()
    cp.wait()

    def group_body(pg, carry):
        base = o * (P * G) + pg * (8 * G)
        accs = []
        for q in range(8):
            acc = jnp.zeros((SLAB, 128), jnp.float32)
            for g in range(G):
                i = pl.multiple_of(idx_ref[base + q * G + g], SLAB)
                acc = acc + emb_vmem[pl.ds(i, SLAB), :]
            accs.append(acc * (1.0 / G))
        for j in range(SLAB):
            blk = jnp.concatenate([a[j:j + 1, :] for a in accs], axis=0)
            protos[pl.ds(j * P + pg * 8, 8), :] = blk
        return carry

    jax.lax.fori_loop(0, GROUPS, group_body, 0)

    h = b1_ref[0]
    h = h + sum(
        jnp.dot(protos[pl.ds(j * P, P), :],
                W1_ref[0, j * 128:(j + 1) * 128, :],
                preferred_element_type=jnp.float32)
        for j in range(SLAB))
    h = jnp.maximum(h, 0.0)
    z = jnp.dot(h, W2_ref[0], preferred_element_type=jnp.float32) + b2_ref[0]
    z_ref[0] = z


def _loss_kernel(z_ref, out_ref):
    M = O * P
    z = z_ref[...]
    nrm = jnp.sqrt(jnp.sum(z * z, axis=1, keepdims=True))
    zn = z / (nrm + EPS)
    sim = jax.lax.dot_general(zn, zn, (((1,), (1,)), ((), ())),
                              preferred_element_type=jnp.float32) * TEMP_INV
    ri = jax.lax.broadcasted_iota(jnp.int32, (M, M), 0)
    ci = jax.lax.broadcasted_iota(jnp.int32, (M, M), 1)
    eye = ri == ci
    pos = ((ri % P) == (ci % P)) & (~eye)
    ex = jnp.where(eye, 0.0, jnp.exp(sim))
    denom = jnp.sum(ex, axis=1) + EPS
    possum = jnp.sum(jnp.where(pos, ex, 0.0), axis=1) + EPS
    out_ref[0, 0] = jnp.mean(jnp.log(denom) - jnp.log(possum))


@jax.jit
def kernel(emb, W1, b1, W2, b2, idx):
    emb2d = emb.reshape(O, N * SLAB, 128)
    idx4 = (idx.astype(jnp.int32) * SLAB).reshape(O * P * G)
    b1r = b1.reshape(O, 1, H)
    b2r = b2.reshape(O, 1, D)

    z = pl.pallas_call(
        _proj_kernel,
        grid_spec=pltpu.PrefetchScalarGridSpec(
            num_scalar_prefetch=1,
            grid=(O,),
            in_specs=[
                pl.BlockSpec(memory_space=pltpu.MemorySpace.HBM),
                pl.BlockSpec((1, H, H), lambda o, idx_ref: (o, 0, 0)),
                pl.BlockSpec((1, 1, H), lambda o, idx_ref: (o, 0, 0)),
                pl.BlockSpec((1, H, D), lambda o, idx_ref: (o, 0, 0)),
                pl.BlockSpec((1, 1, D), lambda o, idx_ref: (o, 0, 0)),
            ],
            out_specs=pl.BlockSpec((1, P, D), lambda o, idx_ref: (o, 0, 0)),
            scratch_shapes=[
                pltpu.VMEM((N * SLAB, 128), jnp.float32),
                pltpu.VMEM((SLAB * P, 128), jnp.float32),
                pltpu.SemaphoreType.DMA,
            ],
        ),
        out_shape=jax.ShapeDtypeStruct((O, P, D), jnp.float32),
        compiler_params=pltpu.CompilerParams(
            dimension_semantics=("parallel",)),
    )(idx4, emb2d, W1, b1r, W2, b2r)

    loss = pl.pallas_call(
        _loss_kernel,
        in_specs=[pl.BlockSpec(memory_space=pltpu.MemorySpace.VMEM)],
        out_specs=pl.BlockSpec(memory_space=pltpu.MemorySpace.SMEM),
        out_shape=jax.ShapeDtypeStruct((1, 1), jnp.float32),
    )(z.reshape(O * P, D))
    return loss[0, 0]


# speedup vs baseline: 1.9309x; 1.9309x over previous
import jax
import jax.numpy as jnp
from jax.experimental import pallas as pl
from jax.experimental.pallas import tpu as pltpu

O, P, G, N, H, D = 3, 128, 64, 20000, 512, 128
TEMP_INV = 10.0
EPS = 1e-12
QP = 4
GROUPS = P // QP


def _proj_kernel(idx_ref, emb_ref, W1_ref, b1_ref, W2_ref, b2_ref,
                 z_ref, emb_vmem, acc3, sem):
    o = pl.program_id(0)
    cp = pltpu.make_async_copy(emb_ref.at[o], emb_vmem, sem)
    cp.start()
    cp.wait()

    iota8 = jax.lax.broadcasted_iota(jnp.int32, (8, H), 0)

    def group_body(pg, carry):
        base = o * (P * G) + pg * (QP * G)
        for q in range(QP):
            acc = jnp.zeros((8, H), jnp.float32)
            for g in range(G):
                i = idx_ref[base + q * G + g]
                ib = pl.multiple_of((i >> 3) << 3, 8)
                chunk = emb_vmem[pl.ds(ib, 8), :]
                acc = acc + jnp.where(iota8 == (i & 7), chunk, 0.0)
            acc3[pg * QP + q] = acc
        return carry

    jax.lax.fori_loop(0, GROUPS, group_body, 0)

    protos = jnp.sum(acc3[...], axis=1) * (1.0 / G)
    h = jnp.dot(protos, W1_ref[0], preferred_element_type=jnp.float32)
    h = jnp.maximum(h + b1_ref[0], 0.0)
    z = jnp.dot(h, W2_ref[0], preferred_element_type=jnp.float32) + b2_ref[0]
    z_ref[0] = z


def _loss_kernel(z_ref, out_ref):
    M = O * P
    z = z_ref[...]
    nrm = jnp.sqrt(jnp.sum(z * z, axis=1, keepdims=True))
    zn = z / (nrm + EPS)
    sim = jax.lax.dot_general(zn, zn, (((1,), (1,)), ((), ())),
                              preferred_element_type=jnp.float32) * TEMP_INV
    ri = jax.lax.broadcasted_iota(jnp.int32, (M, M), 0)
    ci = jax.lax.broadcasted_iota(jnp.int32, (M, M), 1)
    eye = ri == ci
    pos = ((ri % P) == (ci % P)) & (~eye)
    ex = jnp.where(eye, 0.0, jnp.exp(sim))
    denom = jnp.sum(ex, axis=1) + EPS
    possum = jnp.sum(jnp.where(pos, ex, 0.0), axis=1) + EPS
    out_ref[0, 0] = jnp.mean(jnp.log(denom) - jnp.log(possum))


@jax.jit
def kernel(emb, W1, b1, W2, b2, idx):
    idxf = idx.astype(jnp.int32).reshape(O * P * G)
    b1r = b1.reshape(O, 1, H)
    b2r = b2.reshape(O, 1, D)

    z = pl.pallas_call(
        _proj_kernel,
        grid_spec=pltpu.PrefetchScalarGridSpec(
            num_scalar_prefetch=1,
            grid=(O,),
            in_specs=[
                pl.BlockSpec(memory_space=pltpu.MemorySpace.HBM),
                pl.BlockSpec((1, H, H), lambda o, idx_ref: (o, 0, 0)),
                pl.BlockSpec((1, 1, H), lambda o, idx_ref: (o, 0, 0)),
                pl.BlockSpec((1, H, D), lambda o, idx_ref: (o, 0, 0)),
                pl.BlockSpec((1, 1, D), lambda o, idx_ref: (o, 0, 0)),
            ],
            out_specs=pl.BlockSpec((1, P, D), lambda o, idx_ref: (o, 0, 0)),
            scratch_shapes=[
                pltpu.VMEM((N, H), jnp.float32),
                pltpu.VMEM((P, 8, H), jnp.float32),
                pltpu.SemaphoreType.DMA,
            ],
        ),
        out_shape=jax.ShapeDtypeStruct((O, P, D), jnp.float32),
        compiler_params=pltpu.CompilerParams(
            dimension_semantics=("parallel",)),
    )(idxf, emb, W1, b1r, W2, b2r)

    loss = pl.pallas_call(
        _loss_kernel,
        in_specs=[pl.BlockSpec(memory_space=pltpu.MemorySpace.VMEM)],
        out_specs=pl.BlockSpec(memory_space=pltpu.MemorySpace.SMEM),
        out_shape=jax.ShapeDtypeStruct((1, 1), jnp.float32),
    )(z.reshape(O * P, D))
    return loss[0, 0]
